# Initial kernel scaffold; baseline (speedup 1.0000x reference)
import jax
import jax.numpy as jnp
from jax.experimental import pallas as pl


def kernel(x, codebook, values):
    raise NotImplementedError("write your pallas kernel here")


# speedup vs baseline: 1.0211x; 1.0211x over previous
import functools

import jax
import jax.numpy as jnp
from jax import lax
from jax.experimental import pallas as pl
from jax.experimental.pallas import tpu as pltpu
from jax.experimental.pallas import tpu_sc as plsc

B, N, DIM = 16, 256, 384
K = 8192
DIM_MEM = 256
M = B * N

BM = 512
BK = 1024
MB = M // BM
KB = K // BK


def _argmax_body(x_ref, cb_ref, out_ref, mval_ref, midx_ref):
    k = pl.program_id(1)

    cb = cb_ref[...]
    n2 = jnp.sum(cb * cb, axis=1)
    s = 2.0 * lax.dot_general(
        x_ref[...], cb,
        (((1,), (1,)), ((), ())),
        preferred_element_type=jnp.float32,
        precision=lax.Precision.DEFAULT,
    ) - n2[None, :]

    rowmax = jnp.max(s, axis=1, keepdims=True)
    iota = lax.broadcasted_iota(jnp.int32, s.shape, 1)
    rowarg = jnp.min(
        jnp.where(s == rowmax, iota, jnp.int32(2**30)), axis=1, keepdims=True
    ) + k * BK

    @pl.when(k == 0)
    def _init():
        mval_ref[...] = rowmax
        midx_ref[...] = rowarg

    @pl.when(k > 0)
    def _update():
        better = rowmax > mval_ref[...]
        mval_ref[...] = jnp.where(better, rowmax, mval_ref[...])
        midx_ref[...] = jnp.where(better, rowarg, midx_ref[...])

    @pl.when(k == KB - 1)
    def _emit():
        out_ref[...] = midx_ref[...][None]


def _nearest_codes(flatten, codebook, interpret=False):
    out = pl.pallas_call(
        _argmax_body,
        grid=(MB, KB),
        in_specs=[
            pl.BlockSpec((BM, DIM), lambda m, k: (m, 0)),
            pl.BlockSpec((BK, DIM), lambda m, k: (k, 0)),
        ],
        out_specs=pl.BlockSpec((1, BM, 1), lambda m, k: (m, 0, 0)),
        out_shape=jax.ShapeDtypeStruct((MB, BM, 1), jnp.int32),
        scratch_shapes=[
            pltpu.VMEM((BM, 1), jnp.float32),
            pltpu.VMEM((BM, 1), jnp.int32),
        ],
        interpret=interpret,
    )(flatten, codebook)
    return out.reshape(M)


@functools.cache
def _make_gather():
    info = plsc.get_sparse_core_info()
    nc, ns = info.num_cores, info.num_subcores
    rows_per_w = M // (nc * ns)

    def _gather_body(idx_hbm, values_hbm, out_hbm, idx_v, rows_v, sem):
        wid = lax.axis_index("s") * nc + lax.axis_index("c")
        base = wid * rows_per_w
        pltpu.sync_copy(idx_hbm.at[pl.ds(base, rows_per_w)], idx_v)
        pltpu.async_copy(values_hbm.at[idx_v], rows_v, sem).wait()
        pltpu.sync_copy(rows_v, out_hbm.at[pl.ds(base, rows_per_w)])

    return pl.kernel(
        _gather_body,
        out_type=jax.ShapeDtypeStruct((M, DIM_MEM), jnp.float32),
        mesh=plsc.VectorSubcoreMesh(core_axis_name="c", subcore_axis_name="s"),
        scratch_types=[
            pltpu.VMEM((rows_per_w,), jnp.int32),
            pltpu.VMEM((rows_per_w, DIM_MEM), jnp.float32),
            pltpu.SemaphoreType.DMA,
        ],
    )


def kernel(x, codebook, values):
    flatten = x.reshape(M, DIM)
    idx = _nearest_codes(flatten, codebook)
    memories = _make_gather()(idx, values)
    return memories.reshape(B, N, DIM_MEM)
